# baseline (speedup 1.0000x reference)
import functools

import jax
import jax.numpy as jnp
from jax import lax
from jax.experimental import pallas as pl
from jax.experimental.pallas import tpu as pltpu
from jax.experimental.pallas import tpu_sc as plsc


def _sc_embedding_sum(tokens_t, embedding_table):
    seq_len, batch = tokens_t.shape
    _, dim = embedding_table.shape
    info = plsc.get_sparse_core_info()
    num_workers = info.num_cores * info.num_subcores
    b_per_w = batch // num_workers

    mesh = plsc.VectorSubcoreMesh(core_axis_name="c", subcore_axis_name="s")

    @functools.partial(
        pl.kernel,
        mesh=mesh,
        out_type=jax.ShapeDtypeStruct((batch, dim), jnp.float32),
        scratch_types=[
            pltpu.VMEM((seq_len, b_per_w), jnp.int32),
            pltpu.VMEM((b_per_w, dim), jnp.float32),
            pltpu.SemaphoreType.DMA,
        ],
        compiler_params=pltpu.CompilerParams(use_tc_tiling_on_sc=False),
    )
    def sc_kernel(tok_hbm, table_hbm, out_hbm, idx_v, acc_v, sem):
        wid = lax.axis_index("s") * info.num_cores + lax.axis_index("c")
        base = wid * b_per_w

        pltpu.sync_copy(tok_hbm.at[:, pl.ds(base, b_per_w)], idx_v)

        zeros = jnp.zeros((16,), jnp.float32)

        def zero_row(r, _):
            for k in range(dim // 16):
                acc_v[r, pl.ds(k * 16, 16)] = zeros
            return 0

        lax.fori_loop(0, b_per_w, zero_row, 0)

        def fire(t, _):
            pltpu.make_async_copy(
                table_hbm.at[idx_v.at[t]], acc_v, sem
            ).start(add=True)
            return 0

        lax.fori_loop(0, seq_len, fire, 0)

        def drain(t, _):
            pltpu.make_async_copy(
                table_hbm.at[idx_v.at[0]], acc_v, sem
            ).wait()
            return 0

        lax.fori_loop(0, seq_len, drain, 0)

        pltpu.sync_copy(acc_v, out_hbm.at[pl.ds(base, b_per_w), :])

    return sc_kernel(tokens_t, embedding_table)


def _tc_fixup(sums, tokens, embedding_table):
    batch, seq_len = tokens.shape
    _, dim = embedding_table.shape

    def tc_kernel(sum_ref, tok_ref, t0_ref, out_ref):
        mask = (tok_ref[...] > 0).astype(jnp.float32)
        cnt = jnp.sum(mask, axis=1, keepdims=True)
        n_zero = seq_len - cnt
        t0 = t0_ref[0:1, :]
        out_ref[...] = (sum_ref[...] - n_zero * t0) / jnp.maximum(cnt, 1.0)

    return pl.pallas_call(
        tc_kernel,
        out_shape=jax.ShapeDtypeStruct((batch, dim), jnp.float32),
        grid=(1,),
        in_specs=[
            pl.BlockSpec((batch, dim), lambda i: (0, 0)),
            pl.BlockSpec((batch, seq_len), lambda i: (0, 0)),
            pl.BlockSpec((8, dim), lambda i: (0, 0)),
        ],
        out_specs=pl.BlockSpec((batch, dim), lambda i: (0, 0)),
    )(sums, tokens, embedding_table)


@jax.jit
def kernel(tokens, embedding_table):
    tokens_t = tokens.T
    sums = _sc_embedding_sum(tokens_t, embedding_table)
    return _tc_fixup(sums, tokens, embedding_table)
